# Initial kernel scaffold; baseline (speedup 1.0000x reference)
import jax
import jax.numpy as jnp
from jax.experimental import pallas as pl


def kernel(column_value_ids, table):
    raise NotImplementedError("write your pallas kernel here")


# speedup vs baseline: 1.1032x; 1.1032x over previous
import functools

import jax
import jax.numpy as jnp
from jax import lax
from jax.experimental import pallas as pl
from jax.experimental.pallas import tpu as pltpu
from jax.experimental.pallas import tpu_sc as plsc

_INFO = plsc.get_sparse_core_info()
_NC = _INFO.num_cores
_NS = _INFO.num_subcores
_NW = _NC * _NS

_CHUNK = 1600


@functools.lru_cache(maxsize=None)
def _build(total: int, emb_dim: int):
    b_per_w = total // _NW
    n_chunks = b_per_w // _CHUNK
    assert b_per_w % _CHUNK == 0 and _CHUNK % 8 == 0

    mesh = plsc.VectorSubcoreMesh(core_axis_name="c", subcore_axis_name="s")

    @functools.partial(
        pl.kernel,
        mesh=mesh,
        out_type=jax.ShapeDtypeStruct((total, emb_dim), jnp.float32),
        scratch_types=[
            pltpu.VMEM((_CHUNK,), jnp.int32),
            pltpu.VMEM((_CHUNK, emb_dim), jnp.float32),
            pltpu.SemaphoreType.DMA,
        ],
        compiler_params=pltpu.CompilerParams(use_tc_tiling_on_sc=False),
    )
    def gather_kernel(idx_hbm, table_hbm, out_hbm, idx_v, rows_v, sem):
        wid = lax.axis_index("s") * _NC + lax.axis_index("c")
        base = wid * b_per_w
        for i in range(n_chunks):
            off = base + i * _CHUNK
            pltpu.sync_copy(idx_hbm.at[pl.ds(off, _CHUNK)], idx_v)
            pltpu.async_copy(table_hbm.at[idx_v], rows_v, sem).wait()
            pltpu.sync_copy(rows_v, out_hbm.at[pl.ds(off, _CHUNK)])

    return gather_kernel


def kernel(column_value_ids, table):
    batch, x_len = column_value_ids.shape
    emb_dim = table.shape[1]
    idx = column_value_ids.reshape(-1).astype(jnp.int32)
    out = _build(idx.shape[0], emb_dim)(idx, table)
    return out.reshape(batch, x_len, emb_dim)


# speedup vs baseline: 1.1136x; 1.0094x over previous
import functools

import jax
import jax.numpy as jnp
from jax import lax
from jax.experimental import pallas as pl
from jax.experimental.pallas import tpu as pltpu
from jax.experimental.pallas import tpu_sc as plsc

_INFO = plsc.get_sparse_core_info()
_NC = _INFO.num_cores
_NS = _INFO.num_subcores
_NW = _NC * _NS

_CHUNK = 1600


@functools.lru_cache(maxsize=None)
def _build(total: int, emb_dim: int):
    b_per_w = total // _NW
    n_chunks = b_per_w // _CHUNK
    assert b_per_w % _CHUNK == 0 and _CHUNK % 8 == 0

    mesh = plsc.VectorSubcoreMesh(core_axis_name="c", subcore_axis_name="s")

    @functools.partial(
        pl.kernel,
        mesh=mesh,
        out_type=jax.ShapeDtypeStruct((total, emb_dim), jnp.float32),
        scratch_types=[
            pltpu.VMEM((b_per_w,), jnp.int32),
            pltpu.VMEM((2, _CHUNK, emb_dim), jnp.float32),
            pltpu.SemaphoreType.DMA((2,)),
            pltpu.SemaphoreType.DMA((2,)),
        ],
        compiler_params=pltpu.CompilerParams(use_tc_tiling_on_sc=False),
    )
    def gather_kernel(idx_hbm, table_hbm, out_hbm, idx_v, rows_v, gsem, ssem):
        wid = lax.axis_index("s") * _NC + lax.axis_index("c")
        base = wid * b_per_w
        pltpu.sync_copy(idx_hbm.at[pl.ds(base, b_per_w)], idx_v)

        def start_gather(i):
            p = i % 2
            return pltpu.async_copy(
                table_hbm.at[idx_v.at[pl.ds(i * _CHUNK, _CHUNK)]],
                rows_v.at[p],
                gsem.at[p],
            )

        def start_store(i):
            p = i % 2
            return pltpu.async_copy(
                rows_v.at[p],
                out_hbm.at[pl.ds(base + i * _CHUNK, _CHUNK)],
                ssem.at[p],
            )

        gathers = [None] * n_chunks
        stores = [None] * n_chunks
        gathers[0] = start_gather(0)
        for i in range(n_chunks):
            if i + 1 < n_chunks:
                if i >= 1:
                    stores[i - 1].wait()
                gathers[i + 1] = start_gather(i + 1)
            gathers[i].wait()
            stores[i] = start_store(i)
        stores[n_chunks - 2].wait()
        stores[n_chunks - 1].wait()

    return gather_kernel


def kernel(column_value_ids, table):
    batch, x_len = column_value_ids.shape
    emb_dim = table.shape[1]
    idx = column_value_ids.reshape(-1).astype(jnp.int32)
    out = _build(idx.shape[0], emb_dim)(idx, table)
    return out.reshape(batch, x_len, emb_dim)


# speedup vs baseline: 1.5730x; 1.4126x over previous
import functools

import jax
import jax.numpy as jnp
from jax import lax
from jax.experimental import pallas as pl
from jax.experimental.pallas import tpu as pltpu
from jax.experimental.pallas import tpu_sc as plsc

_INFO = plsc.get_sparse_core_info()
_NC = _INFO.num_cores
_NS = _INFO.num_subcores
_NW = _NC * _NS

_L = 128


@functools.lru_cache(maxsize=None)
def _build(batch: int, x_len: int, emb_dim: int):
    n_btiles = batch // _L
    bt_per_w = n_btiles // _NW
    n_gtiles = emb_dim // 8
    slab = bt_per_w * _L * x_len
    units = bt_per_w * x_len
    out_words = x_len * n_gtiles * n_btiles * 8 * _L
    assert batch % (_L * _NW) == 0 and emb_dim % 8 == 0 and units % 2 == 0

    mesh = plsc.VectorSubcoreMesh(core_axis_name="c", subcore_axis_name="s")

    @functools.partial(
        pl.kernel,
        mesh=mesh,
        out_type=jax.ShapeDtypeStruct((out_words,), jnp.float32),
        scratch_types=[
            pltpu.VMEM((slab,), jnp.int32),
            pltpu.VMEM((2, _L), jnp.int32),
            pltpu.VMEM((2, _L, emb_dim), jnp.float32),
            pltpu.VMEM((2, emb_dim * _L), jnp.float32),
            pltpu.SemaphoreType.DMA((2,)),
            pltpu.SemaphoreType.DMA((2,)),
        ],
        compiler_params=pltpu.CompilerParams(
            use_tc_tiling_on_sc=False, needs_layout_passes=False
        ),
    )
    def gather_kernel(idx_hbm, table_hbm, out_hbm, idx_v, gidx_v, rows_v,
                      tile_v, gsem, ssem):
        wid = lax.axis_index("s") * _NC + lax.axis_index("c")
        pltpu.sync_copy(idx_hbm.at[pl.ds(wid * slab, slab)], idx_v)
        lanes = lax.iota(jnp.int32, 16)
        lanes_x = lanes * x_len
        row_idx = [lanes + 16 * j for j in range(_L // 16)]
        col_idx = [lanes * 0 + d for d in range(emb_dim)]

        def split_unit(u):
            bi = u // x_len
            x = u - bi * x_len
            return bi, x

        def build_gidx(u, p):
            bi, x = split_unit(u)
            base = bi * (_L * x_len) + x
            for j in range(_L // 16):
                v = plsc.load_gather(idx_v, [lanes_x + (base + 16 * j * x_len)])
                gidx_v[p, pl.ds(16 * j, 16)] = v

        def start_gather(p):
            pltpu.async_copy(
                table_hbm.at[gidx_v.at[p]], rows_v.at[p], gsem.at[p]
            )

        def wait_gather(p):
            pltpu.make_async_copy(
                table_hbm.at[gidx_v.at[p]], rows_v.at[p], gsem.at[p]
            ).wait()

        def transpose_and_store(u, p):
            rows = rows_v.at[p]
            for d in range(emb_dim):
                for j in range(_L // 16):
                    v = plsc.load_gather(rows, [row_idx[j], col_idx[d]])
                    tile_v[p, pl.ds(d * _L + 16 * j, 16)] = v
            bi, x = split_unit(u)
            bt = wid * bt_per_w + bi
            for g in range(n_gtiles):
                off = ((x * n_gtiles + g) * n_btiles + bt) * (8 * _L)
                pltpu.async_copy(
                    tile_v.at[p, pl.ds(g * 8 * _L, 8 * _L)],
                    out_hbm.at[pl.ds(off, 8 * _L)],
                    ssem.at[p],
                )

        def wait_stores(p):
            for g in range(n_gtiles):
                pltpu.make_async_copy(
                    tile_v.at[p, pl.ds(g * 8 * _L, 8 * _L)],
                    out_hbm.at[pl.ds(0, 8 * _L)],
                    ssem.at[p],
                ).wait()

        build_gidx(0, 0)
        start_gather(0)

        def body(t, _):
            u0 = 2 * t

            @pl.when(t > 0)
            def _():
                wait_stores(0)
            build_gidx(u0 + 1, 1)
            start_gather(1)
            wait_gather(0)
            transpose_and_store(u0, 0)

            @pl.when(t > 0)
            def _():
                wait_stores(1)

            @pl.when(t + 1 < units // 2)
            def _():
                build_gidx(u0 + 2, 0)
                start_gather(0)
            wait_gather(1)
            transpose_and_store(u0 + 1, 1)
            return 0

        lax.fori_loop(0, units // 2, body, 0)
        wait_stores(0)
        wait_stores(1)

    return gather_kernel


def kernel(column_value_ids, table):
    batch, x_len = column_value_ids.shape
    emb_dim = table.shape[1]
    n_gtiles = emb_dim // 8
    idx = column_value_ids.reshape(-1).astype(jnp.int32)
    out5 = _build(batch, x_len, emb_dim)(idx, table)
    out5 = out5.reshape(x_len, n_gtiles, batch // _L, 8, _L)
    out = out5.transpose(2, 4, 0, 1, 3).reshape(batch, x_len, emb_dim)
    return out


# speedup vs baseline: 1.9139x; 1.2167x over previous
import functools

import jax
import jax.numpy as jnp
from jax import lax
from jax.experimental import pallas as pl
from jax.experimental.pallas import tpu as pltpu
from jax.experimental.pallas import tpu_sc as plsc

_INFO = plsc.get_sparse_core_info()
_NC = _INFO.num_cores
_NS = _INFO.num_subcores
_NW = _NC * _NS

_L = 128


@functools.lru_cache(maxsize=None)
def _build(batch: int, x_len: int, emb_dim: int):
    n_btiles = batch // _L
    bt_per_w = n_btiles // _NW
    n_gtiles = emb_dim // 8
    slab = bt_per_w * _L * x_len
    units = bt_per_w * x_len
    out_words = x_len * n_gtiles * n_btiles * 8 * _L
    assert batch % (_L * _NW) == 0 and emb_dim % 8 == 0 and units % 2 == 0

    mesh = plsc.VectorSubcoreMesh(core_axis_name="c", subcore_axis_name="s")

    @functools.partial(
        pl.kernel,
        mesh=mesh,
        out_type=jax.ShapeDtypeStruct((out_words,), jnp.float32),
        scratch_types=[
            pltpu.VMEM((slab,), jnp.int32),
            pltpu.VMEM((2, _L), jnp.int32),
            pltpu.VMEM((2, _L, emb_dim), jnp.float32),
            pltpu.VMEM((2, emb_dim * _L), jnp.float32),
            pltpu.SemaphoreType.DMA((2,)),
            pltpu.SemaphoreType.DMA((2,)),
        ],
        compiler_params=pltpu.CompilerParams(
            use_tc_tiling_on_sc=False, needs_layout_passes=False
        ),
    )
    def gather_kernel(idx_hbm, table_hbm, out_hbm, idx_v, gidx_v, rows_v,
                      tile_v, gsem, ssem):
        wid = lax.axis_index("s") * _NC + lax.axis_index("c")
        pltpu.sync_copy(idx_hbm.at[pl.ds(wid * slab, slab)], idx_v)
        lanes = lax.iota(jnp.int32, 16)
        lanes_x = lanes * x_len
        mod16 = [lax.rem(lanes + s, 16) for s in range(16)]
        diag_s = [mod16[s] * _L + lanes for s in range(16)]
        row16 = [lanes + l0 for l0 in range(0, _L, 16)]

        def split_unit(u):
            bi = u // x_len
            x = u - bi * x_len
            return bi, x

        def build_gidx(u, p):
            bi, x = split_unit(u)
            base = bi * (_L * x_len) + x
            for j in range(_L // 16):
                v = plsc.load_gather(idx_v, [lanes_x + (base + 16 * j * x_len)])
                gidx_v[p, pl.ds(16 * j, 16)] = v

        def start_gather(p):
            pltpu.async_copy(
                table_hbm.at[gidx_v.at[p]], rows_v.at[p], gsem.at[p]
            )

        def wait_gather(p):
            pltpu.make_async_copy(
                table_hbm.at[gidx_v.at[p]], rows_v.at[p], gsem.at[p]
            ).wait()

        def transpose_and_store(u, p):
            rows = rows_v.at[p]
            tile = tile_v.at[p]
            for k in range(_L // 16):
                for d0 in range(0, emb_dim, 16):
                    for s in range(16):
                        v = plsc.load_gather(rows, [row16[k], mod16[s] + d0])
                        plsc.store_scatter(
                            tile, [diag_s[s] + (d0 * _L + 16 * k)], v
                        )
            bi, x = split_unit(u)
            bt = wid * bt_per_w + bi
            for g in range(n_gtiles):
                off = ((x * n_gtiles + g) * n_btiles + bt) * (8 * _L)
                pltpu.async_copy(
                    tile_v.at[p, pl.ds(g * 8 * _L, 8 * _L)],
                    out_hbm.at[pl.ds(off, 8 * _L)],
                    ssem.at[p],
                )

        def wait_stores(p):
            for g in range(n_gtiles):
                pltpu.make_async_copy(
                    tile_v.at[p, pl.ds(g * 8 * _L, 8 * _L)],
                    out_hbm.at[pl.ds(0, 8 * _L)],
                    ssem.at[p],
                ).wait()

        build_gidx(0, 0)
        start_gather(0)

        def body(t, _):
            u0 = 2 * t

            @pl.when(t > 0)
            def _():
                wait_stores(0)
            build_gidx(u0 + 1, 1)
            start_gather(1)
            wait_gather(0)
            transpose_and_store(u0, 0)

            @pl.when(t > 0)
            def _():
                wait_stores(1)

            @pl.when(t + 1 < units // 2)
            def _():
                build_gidx(u0 + 2, 0)
                start_gather(0)
            wait_gather(1)
            transpose_and_store(u0 + 1, 1)
            return 0

        lax.fori_loop(0, units // 2, body, 0)
        wait_stores(0)
        wait_stores(1)

    return gather_kernel


def kernel(column_value_ids, table):
    batch, x_len = column_value_ids.shape
    emb_dim = table.shape[1]
    n_gtiles = emb_dim // 8
    idx = column_value_ids.reshape(-1).astype(jnp.int32)
    out5 = _build(batch, x_len, emb_dim)(idx, table)
    out5 = out5.reshape(x_len, n_gtiles, batch // _L, 8, _L)
    out = out5.transpose(2, 4, 0, 1, 3).reshape(batch, x_len, emb_dim)
    return out
